# Initial kernel scaffold; baseline (speedup 1.0000x reference)
import jax
import jax.numpy as jnp
from jax.experimental import pallas as pl


def kernel(input_, gumbel_u, weight_ih, weight_hh, bias, weight_im, weight_hm, weight_um, fc1_w, fc1_b, fc2_w, fc2_b):
    raise NotImplementedError("write your pallas kernel here")


# speedup vs baseline: 21.5629x; 21.5629x over previous
import functools

import jax
import jax.numpy as jnp
from jax.experimental import pallas as pl
from jax.experimental.pallas import tpu as pltpu

T, B, D, H, MC, ES = 32, 64, 256, 256, 1024, 256


def _step_kernel(x_ref, g_ref, wih_ref, whh_ref, bih_ref, bhh_ref,
                 wim_ref, whm_ref, wum_ref, fc1w_ref, fc1b_ref,
                 fc2a_ref, fc2b_ref, fc2bias_ref,
                 out_ref, h_scr, lu_scr):
    t = pl.program_id(0)

    @pl.when(t == 0)
    def _init():
        h_scr[...] = jnp.zeros((B, H), jnp.float32)
        lu_scr[...] = jnp.full((B, MC), -99999.0, jnp.float32)
        out_ref[...] = jnp.zeros((T, B, H), jnp.float32)

    x = x_ref[0]
    h = h_scr[...]
    lu = lu_scr[...]

    last_use = jax.nn.sigmoid(lu)
    pre = jnp.tanh(jnp.dot(x, wim_ref[...])
                   + jnp.dot(h, whm_ref[...])
                   + jnp.dot(last_use, wum_ref[...]))
    read_head = jnp.dot(pre, fc1w_ref[...]) + fc1b_ref[...]
    u = g_ref[0]
    g = -jnp.log(1e-20 - jnp.log(1e-20 + u))
    logits = read_head + g

    m = jnp.max(logits, axis=1, keepdims=True)
    col = jax.lax.broadcasted_iota(jnp.int32, (B, MC), 1)
    pos = jnp.min(jnp.where(logits == m, col, MC), axis=1)

    row = jax.lax.broadcasted_iota(jnp.int32, (T, B), 0)
    sel = jnp.where((row == (pos[None, :] - 1)) & (row < t), 1.0, 0.0)
    entry = jnp.sum(sel[:, :, None] * out_ref[...], axis=0)

    lu_scr[...] = jnp.where(col == pos[:, None], -1.0, lu - 1.0)

    h_new = (jnp.dot(entry, fc2a_ref[...]) + jnp.dot(h, fc2b_ref[...])
             + fc2bias_ref[...])

    wi = jnp.dot(x, wih_ref[...]) + bih_ref[...]
    wh = jnp.dot(h_new, whh_ref[...]) + bhh_ref[...]
    r = jax.nn.sigmoid(wi[:, :H] + wh[:, :H])
    z = jax.nn.sigmoid(wi[:, H:2 * H] + wh[:, H:2 * H])
    n = jnp.tanh(wi[:, 2 * H:] + r * wh[:, 2 * H:])
    h_out = (1.0 - z) * n + z * h_new

    h_scr[...] = h_out
    out_ref[t] = h_out


@functools.partial(jax.jit, static_argnames=())
def kernel(input_, gumbel_u, weight_ih, weight_hh, bias, weight_im,
           weight_hm, weight_um, fc1_w, fc1_b, fc2_w, fc2_b):
    bias_ih = bias[: 3 * H].reshape(1, 3 * H)
    bias_hh = bias[3 * H:].reshape(1, 3 * H)
    fc1b = fc1_b.reshape(1, MC)
    fc2bias = fc2_b.reshape(1, H)
    fc2a = fc2_w[:ES]
    fc2b = fc2_w[ES:]

    full = lambda shape: pl.BlockSpec(shape, lambda t: (0,) * len(shape))
    return pl.pallas_call(
        _step_kernel,
        grid=(T,),
        in_specs=[
            pl.BlockSpec((1, B, D), lambda t: (t, 0, 0)),
            pl.BlockSpec((1, B, MC), lambda t: (t, 0, 0)),
            full((D, 3 * H)),
            full((H, 3 * H)),
            full((1, 3 * H)),
            full((1, 3 * H)),
            full((D, ES)),
            full((H, ES)),
            full((MC, ES)),
            full((ES, MC)),
            full((1, MC)),
            full((ES, H)),
            full((H, H)),
            full((1, H)),
        ],
        out_specs=pl.BlockSpec((T, B, H), lambda t: (0, 0, 0)),
        out_shape=jax.ShapeDtypeStruct((T, B, H), jnp.float32),
        scratch_shapes=[
            pltpu.VMEM((B, H), jnp.float32),
            pltpu.VMEM((B, MC), jnp.float32),
        ],
        compiler_params=pltpu.CompilerParams(
            dimension_semantics=("arbitrary",),
        ),
    )(input_, gumbel_u, weight_ih, weight_hh, bias_ih, bias_hh,
      weight_im, weight_hm, weight_um, fc1_w, fc1b, fc2a, fc2b, fc2bias)


# speedup vs baseline: 24.8283x; 1.1514x over previous
import functools

import jax
import jax.numpy as jnp
from jax.experimental import pallas as pl
from jax.experimental.pallas import tpu as pltpu

T, B, D, H, MC, ES = 32, 64, 256, 256, 1024, 256


def _step_kernel(x_ref, g_ref, wih_ref, whh_ref, bih_ref, bhh_ref,
                 wim_ref, whm_ref, wum_ref, fc1w_ref, fc1b_ref,
                 fc2a_ref, fc2b_ref, fc2bias_ref,
                 out_ref, h_scr, lu_scr):
    t = pl.program_id(0)

    @pl.when(t == 0)
    def _init():
        h_scr[...] = jnp.zeros((B, H), jnp.float32)
        lu_scr[...] = jnp.full((B, MC), -99999.0, jnp.float32)

    x = x_ref[0]
    h = h_scr[...]
    lu = lu_scr[...]

    last_use = jax.nn.sigmoid(lu)
    pre = jnp.tanh(jnp.dot(x, wim_ref[...])
                   + jnp.dot(h, whm_ref[...])
                   + jnp.dot(last_use, wum_ref[...]))
    read_head = jnp.dot(pre, fc1w_ref[...]) + fc1b_ref[...]
    u = g_ref[0]
    g = -jnp.log(1e-20 - jnp.log(1e-20 + u))
    logits = read_head + g

    m = jnp.max(logits, axis=1, keepdims=True)
    col = jax.lax.broadcasted_iota(jnp.int32, (B, MC), 1)
    pos = jnp.min(jnp.where(logits == m, col, MC), axis=1,
                  keepdims=True)

    idx = jnp.clip(pos - 1, 0, T - 1)
    nodes = [out_ref[s] for s in range(T)]
    for level in range(5):
        take_hi = ((idx >> level) & 1) == 1
        nodes = [jnp.where(take_hi, nodes[2 * i + 1], nodes[2 * i])
                 for i in range(len(nodes) // 2)]
    valid = (pos >= 1) & (pos <= t)
    entry = jnp.where(valid, nodes[0], 0.0)

    lu_scr[...] = jnp.where(col == pos, -1.0, lu - 1.0)

    h_new = (jnp.dot(entry, fc2a_ref[...]) + jnp.dot(h, fc2b_ref[...])
             + fc2bias_ref[...])

    wi = jnp.dot(x, wih_ref[...]) + bih_ref[...]
    wh = jnp.dot(h_new, whh_ref[...]) + bhh_ref[...]
    r = jax.nn.sigmoid(wi[:, :H] + wh[:, :H])
    z = jax.nn.sigmoid(wi[:, H:2 * H] + wh[:, H:2 * H])
    n = jnp.tanh(wi[:, 2 * H:] + r * wh[:, 2 * H:])
    h_out = (1.0 - z) * n + z * h_new

    h_scr[...] = h_out
    out_ref[t] = h_out


@functools.partial(jax.jit, static_argnames=())
def kernel(input_, gumbel_u, weight_ih, weight_hh, bias, weight_im,
           weight_hm, weight_um, fc1_w, fc1_b, fc2_w, fc2_b):
    bias_ih = bias[: 3 * H].reshape(1, 3 * H)
    bias_hh = bias[3 * H:].reshape(1, 3 * H)
    fc1b = fc1_b.reshape(1, MC)
    fc2bias = fc2_b.reshape(1, H)
    fc2a = fc2_w[:ES]
    fc2b = fc2_w[ES:]

    full = lambda shape: pl.BlockSpec(shape, lambda t: (0,) * len(shape))
    return pl.pallas_call(
        _step_kernel,
        grid=(T,),
        in_specs=[
            pl.BlockSpec((1, B, D), lambda t: (t, 0, 0)),
            pl.BlockSpec((1, B, MC), lambda t: (t, 0, 0)),
            full((D, 3 * H)),
            full((H, 3 * H)),
            full((1, 3 * H)),
            full((1, 3 * H)),
            full((D, ES)),
            full((H, ES)),
            full((MC, ES)),
            full((ES, MC)),
            full((1, MC)),
            full((ES, H)),
            full((H, H)),
            full((1, H)),
        ],
        out_specs=pl.BlockSpec((T, B, H), lambda t: (0, 0, 0)),
        out_shape=jax.ShapeDtypeStruct((T, B, H), jnp.float32),
        scratch_shapes=[
            pltpu.VMEM((B, H), jnp.float32),
            pltpu.VMEM((B, MC), jnp.float32),
        ],
        compiler_params=pltpu.CompilerParams(
            dimension_semantics=("arbitrary",),
        ),
    )(input_, gumbel_u, weight_ih, weight_hh, bias_ih, bias_hh,
      weight_im, weight_hm, weight_um, fc1_w, fc1b, fc2a, fc2b, fc2bias)
